# Initial kernel scaffold; baseline (speedup 1.0000x reference)
import jax
import jax.numpy as jnp
from jax.experimental import pallas as pl


def kernel(x, edge_index, W1, b1, W2, b2):
    raise NotImplementedError("write your pallas kernel here")


# speedup vs baseline: 20.3407x; 20.3407x over previous
import functools

import jax
import jax.numpy as jnp
from jax import lax
from jax.experimental import pallas as pl
from jax.experimental.pallas import tpu as pltpu
from jax.experimental.pallas import tpu_sc as plsc

N = 10000
NP = 10240
D = 128
E = 320000
NC = 2
NS = 16
NW = NC * NS
EPT = E // NW
CHUNK = 125
NCHUNK = EPT // CHUNK
ROWS = NP // NS
DEGW = 16

_mesh = plsc.VectorSubcoreMesh(
    core_axis_name="c", subcore_axis_name="s", num_cores=NC, num_subcores=NS
)


@functools.partial(
    pl.kernel,
    out_type=jax.ShapeDtypeStruct((NC, NP, DEGW), jnp.float32),
    mesh=_mesh,
    scratch_types=[
        pltpu.VMEM((NCHUNK, CHUNK), jnp.int32),
        pltpu.VMEM((CHUNK, DEGW), jnp.float32),
        pltpu.VMEM_SHARED((NP, DEGW), jnp.float32),
    ],
)
def _deg_kernel(dst_hbm, ones_hbm, out_hbm, dst_v, ones_v, acc):
    cid = lax.axis_index("c")
    sid = lax.axis_index("s")
    wid = cid * NS + sid
    r0 = sid * ROWS
    pltpu.sync_copy(ones_hbm.at[pl.ds(r0, ROWS)], acc.at[pl.ds(r0, ROWS)])
    for r in range(CHUNK):
        ones_v[r, :] = jnp.full((DEGW,), 1.0, jnp.float32)
    pltpu.sync_copy(dst_hbm.at[pl.ds(wid * NCHUNK, NCHUNK)], dst_v)
    plsc.subcore_barrier()

    def body(i, carry):
        pltpu.sync_copy(ones_v, acc.at[dst_v.at[i]], add=True)
        return carry

    lax.fori_loop(0, NCHUNK, body, 0)
    plsc.subcore_barrier()
    pltpu.sync_copy(acc.at[pl.ds(r0, ROWS)], out_hbm.at[cid, pl.ds(r0, ROWS)])


@functools.partial(
    pl.kernel,
    out_type=jax.ShapeDtypeStruct((NC, NP, D), jnp.float32),
    mesh=_mesh,
    scratch_types=[
        pltpu.VMEM((NCHUNK, CHUNK), jnp.int32),
        pltpu.VMEM((NCHUNK, CHUNK), jnp.int32),
        pltpu.VMEM((CHUNK, D), jnp.float32),
        pltpu.VMEM_SHARED((NP, D), jnp.float32),
        pltpu.SemaphoreType.DMA,
    ],
)
def _mp_kernel(g_hbm, src_hbm, dst_hbm, out_hbm, src_v, dst_v, rows_v, acc, sem):
    cid = lax.axis_index("c")
    sid = lax.axis_index("s")
    wid = cid * NS + sid
    r0 = sid * ROWS
    pltpu.sync_copy(g_hbm.at[pl.ds(r0, ROWS)], acc.at[pl.ds(r0, ROWS)])
    pltpu.sync_copy(src_hbm.at[pl.ds(wid * NCHUNK, NCHUNK)], src_v)
    pltpu.sync_copy(dst_hbm.at[pl.ds(wid * NCHUNK, NCHUNK)], dst_v)
    plsc.subcore_barrier()

    def body(i, carry):
        pltpu.async_copy(g_hbm.at[src_v.at[i]], rows_v, sem).wait()
        pltpu.sync_copy(rows_v, acc.at[dst_v.at[i]], add=True)
        return carry

    lax.fori_loop(0, NCHUNK, body, 0)
    plsc.subcore_barrier()
    pltpu.sync_copy(acc.at[pl.ds(r0, ROWS)], out_hbm.at[cid, pl.ds(r0, ROWS)])


def _mm_body(x_ref, w_ref, o_ref):
    o_ref[...] = jnp.dot(x_ref[...], w_ref[...], preferred_element_type=jnp.float32)


_mm = pl.pallas_call(
    _mm_body, out_shape=jax.ShapeDtypeStruct((NP, D), jnp.float32)
)


def _scale_body(h_ref, degp_ref, g_ref, dinv_ref):
    deg = degp_ref[0] + degp_ref[1] - 1.0
    dinv = lax.rsqrt(deg)[:, 0:1]
    dinv_ref[...] = dinv
    g_ref[...] = h_ref[...] * dinv


_scale = pl.pallas_call(
    _scale_body,
    out_shape=(
        jax.ShapeDtypeStruct((NP, D), jnp.float32),
        jax.ShapeDtypeStruct((NP, 1), jnp.float32),
    ),
)


def _layer2_body(p_ref, g1_ref, dinv_ref, w2_ref, b1_ref, g2_ref):
    s1 = p_ref[0] + p_ref[1] - g1_ref[...]
    x1 = jnp.maximum(s1 * dinv_ref[...] + b1_ref[...], 0.0)
    g2_ref[...] = jnp.dot(
        x1, w2_ref[...], preferred_element_type=jnp.float32
    ) * dinv_ref[...]


_layer2 = pl.pallas_call(
    _layer2_body, out_shape=jax.ShapeDtypeStruct((NP, D), jnp.float32)
)


def _final_body(q_ref, g2_ref, dinv_ref, b2_ref, x_ref, o_ref):
    s2 = q_ref[0] + q_ref[1] - g2_ref[...]
    o_ref[...] = s2 * dinv_ref[...] + b2_ref[...] + x_ref[...]


_final = pl.pallas_call(
    _final_body, out_shape=jax.ShapeDtypeStruct((NP, D), jnp.float32)
)


def kernel(x, edge_index, W1, b1, W2, b2):
    src = edge_index[0].astype(jnp.int32).reshape(NW * NCHUNK, CHUNK)
    dst = edge_index[1].astype(jnp.int32).reshape(NW * NCHUNK, CHUNK)
    ones = jnp.ones((NP, DEGW), jnp.float32)

    xp = jnp.pad(x, ((0, NP - N), (0, 0)))
    degp = _deg_kernel(dst, ones)
    h1 = _mm(xp, W1)
    g1, dinv = _scale(h1, degp)
    p = _mp_kernel(g1, src, dst)
    g2 = _layer2(p, g1, dinv, W2, b1)
    q = _mp_kernel(g2, src, dst)
    return _final(q, g2, dinv, b2, xp)[:N]
